# baseline (speedup 1.0000x reference)
import jax
import jax.numpy as jnp
from jax.experimental import pallas as pl
from jax.experimental.pallas import tpu as pltpu

_BM = 1024


def _probe(x_ref, b_ref, o_ref):
    o_ref[...] = x_ref[:, :64].astype(jnp.float32) + b_ref[...]


def kernel(x, W, b):
    n_tokens, d_model = x.shape
    n_experts = W.shape[0]
    b2 = b.reshape(1, n_experts)
    xi8 = jax.lax.bitcast_convert_type(x, jnp.int8).reshape(n_tokens, 4 * d_model)
    return pl.pallas_call(
        _probe,
        grid=(n_tokens // _BM,),
        in_specs=[
            pl.BlockSpec((_BM, 4 * d_model), lambda i: (i, 0)),
            pl.BlockSpec((1, n_experts), lambda i: (0, 0)),
        ],
        out_specs=pl.BlockSpec((_BM, n_experts), lambda i: (i, 0)),
        out_shape=jax.ShapeDtypeStruct((n_tokens, n_experts), jnp.float32),
        compiler_params=pltpu.CompilerParams(
            vmem_limit_bytes=120 * 1024 * 1024,
        ),
    )(xi8, b2)


# speedup vs baseline: 11.0148x; 11.0148x over previous
import jax
import jax.numpy as jnp
from jax.experimental import pallas as pl
from jax.experimental.pallas import tpu as pltpu

_BM = 1024


def _probe(x_ref, b_ref, o_ref):
    o_ref[...] = x_ref[:, 0, :64] + b_ref[...]


def kernel(x, W, b):
    n_tokens, d_model = x.shape
    n_experts = W.shape[0]
    b2 = b.reshape(1, n_experts)
    x3 = x.reshape(n_tokens, d_model // 128, 128)
    return pl.pallas_call(
        _probe,
        grid=(n_tokens // _BM,),
        in_specs=[
            pl.BlockSpec((_BM, d_model // 128, 128), lambda i: (i, 0, 0)),
            pl.BlockSpec((1, n_experts), lambda i: (0, 0)),
        ],
        out_specs=pl.BlockSpec((_BM, n_experts), lambda i: (i, 0)),
        out_shape=jax.ShapeDtypeStruct((n_tokens, n_experts), jnp.float32),
        compiler_params=pltpu.CompilerParams(
            vmem_limit_bytes=120 * 1024 * 1024,
        ),
    )(x3, b2)


# speedup vs baseline: 35.3808x; 3.2121x over previous
import jax
import jax.numpy as jnp
from jax.experimental import pallas as pl
from jax.experimental.pallas import tpu as pltpu

_BM = 1024


def _router_block(x_ref, w_ref, b_ref, o_ref):
    acc = jax.lax.dot_general(
        x_ref[...].astype(jnp.bfloat16),
        w_ref[...].astype(jnp.bfloat16),
        dimension_numbers=(((1,), (1,)), ((), ())),
        preferred_element_type=jnp.float32,
    )
    o_ref[...] = acc + b_ref[...]


def kernel(x, W, b):
    n_tokens, d_model = x.shape
    n_experts = W.shape[0]
    b2 = b.reshape(1, n_experts)
    return pl.pallas_call(
        _router_block,
        grid=(n_tokens // _BM,),
        in_specs=[
            pl.BlockSpec((_BM, d_model), lambda i: (i, 0)),
            pl.BlockSpec((n_experts, d_model), lambda i: (0, 0)),
            pl.BlockSpec((1, n_experts), lambda i: (0, 0)),
        ],
        out_specs=pl.BlockSpec((_BM, n_experts), lambda i: (i, 0)),
        out_shape=jax.ShapeDtypeStruct((n_tokens, n_experts), jnp.float32),
        compiler_params=pltpu.CompilerParams(
            vmem_limit_bytes=120 * 1024 * 1024,
            dimension_semantics=("arbitrary",),
        ),
    )(x, W, b2)
